# baseline (speedup 1.0000x reference)
import functools

import jax
import jax.numpy as jnp
from jax import lax
from jax.experimental import pallas as pl
from jax.experimental.pallas import tpu as pltpu
from jax.experimental.pallas import tpu_sc as plsc

_NC = 2
_NS = 16
_NW = _NC * _NS

_D = 768
_LANES = 16
_DL = _D // _LANES
_L_SEQ = 4096
_B = 4
_POS_PER_W = _L_SEQ // _NW
_CHUNK = 32
_PC = _POS_PER_W // _CHUNK
_NG = _PC * _B
_NBUF = 3


def _emb_body(idx_hbm, table_hbm, pos_hbm, out_hbm,
              idx_v, pos0, pos1, r0, r1, r2,
              g0, g1, g2, o0, o1, o2, p0, p1):
    pos_v = [pos0, pos1]
    rows = [r0, r1, r2]
    gsem = [g0, g1, g2]
    osem = [o0, o1, o2]
    psem = [p0, p1]
    wid = lax.axis_index("s") * _NC + lax.axis_index("c")
    pos_base = wid * _POS_PER_W
    idesc = [pltpu.async_copy(idx_hbm.at[b, pl.ds(pos_base, _POS_PER_W)],
                              idx_v.at[b], p0)
             for b in range(_B)]

    def start_gather(k):
        c, b = divmod(k, _B)
        return pltpu.async_copy(
            table_hbm.at[idx_v.at[b, pl.ds(c * _CHUNK, _CHUNK)]],
            rows[k % _NBUF], gsem[k % _NBUF])

    def start_pos(c):
        return pltpu.async_copy(
            pos_hbm.at[pl.ds(pos_base + c * _CHUNK, _CHUNK)],
            pos_v[c % 2], psem[c % 2])

    pdesc = {0: start_pos(0)}
    for d in idesc:
        d.wait()
    gdesc = {0: start_gather(0)}
    odesc = {}
    for k in range(_NG):
        c, b = divmod(k, _B)
        if b == 0 and c + 1 < _PC:
            pdesc[c + 1] = start_pos(c + 1)
        if b == 0:
            pdesc[c].wait()
        gdesc[k].wait()
        if k + 1 < _NG:
            if k - 2 >= 0:
                for d in odesc[k - 2]:
                    d.wait()
            gdesc[k + 1] = start_gather(k + 1)
        rv, pv = rows[k % _NBUF], pos_v[c % 2]
        out_base = b * _L_SEQ + pos_base + c * _CHUNK
        half = _CHUNK // 4

        def half_body(h, carry):
            def row_body(r, cr):
                for cc in range(_DL):
                    sl = pl.ds(cc * _LANES, _LANES)
                    plsc.addupdate(rv.at[r, sl], pv[r, sl])
                return cr

            lax.fori_loop(h * half, (h + 1) * half, row_body, 0)
            pltpu.async_copy(
                rv.at[pl.ds(h * half, half)],
                out_hbm.at[pl.ds(out_base + h * half, half)],
                osem[k % _NBUF])
            return carry

        lax.fori_loop(0, 4, half_body, 0)
        odesc[k] = [pltpu.make_async_copy(
            rv.at[pl.ds(h * half, half)],
            out_hbm.at[pl.ds(out_base + h * half, half)],
            osem[k % _NBUF]) for h in range(4)]
    for k in range(_NG - 3, _NG):
        for d in odesc[k]:
            d.wait()


@functools.partial(jax.jit, static_argnames=())
def kernel(x, emb_table, pos_encoding):
    seq_len = x.shape[1]
    idx = x.astype(jnp.int32)
    mesh = plsc.VectorSubcoreMesh(
        core_axis_name="c", subcore_axis_name="s",
        num_cores=_NC, num_subcores=_NS,
    )
    out = pl.kernel(
        _emb_body,
        out_type=jax.ShapeDtypeStruct((_B * _L_SEQ, _D), jnp.float32),
        mesh=mesh,
        scratch_types=(
            [pltpu.VMEM((_B, _POS_PER_W), jnp.int32)]
            + [pltpu.VMEM((_CHUNK, _D), jnp.float32)] * 2
            + [pltpu.VMEM((_CHUNK, _D), jnp.float32)] * _NBUF
            + [pltpu.SemaphoreType.DMA] * 8
        ),
    )(idx, emb_table, pos_encoding)
    return out.reshape(_B, seq_len, _D)


# speedup vs baseline: 1.4126x; 1.4126x over previous
import functools

import jax
import jax.numpy as jnp
from jax import lax
from jax.experimental import pallas as pl
from jax.experimental.pallas import tpu as pltpu
from jax.experimental.pallas import tpu_sc as plsc

_NC = 2
_NS = 16
_NW = _NC * _NS

_D = 768
_LANES = 16
_DL = _D // _LANES
_L_SEQ = 4096
_B = 4
_POS_PER_W = _L_SEQ // _NW
_CHUNK = 32
_PC = _POS_PER_W // _CHUNK
_NG = _PC * _B
_NBUF = 3


def _emb_body(idx_hbm, table_hbm, pos_hbm, out_hbm,
              idx_v, pos0, pos1, r0, r1, r2,
              g0, g1, g2, o0, o1, o2, p0, p1):
    pos_v = [pos0, pos1]
    rows = [r0, r1, r2]
    gsem = [g0, g1, g2]
    osem = [o0, o1, o2]
    psem = [p0, p1]
    wid = lax.axis_index("s") * _NC + lax.axis_index("c")
    pos_base = wid * _POS_PER_W
    idesc = [pltpu.async_copy(idx_hbm.at[b, pl.ds(pos_base, _POS_PER_W)],
                              idx_v.at[b], p0)
             for b in range(_B)]

    def start_gather(k):
        c, b = divmod(k, _B)
        return pltpu.async_copy(
            table_hbm.at[idx_v.at[b, pl.ds(c * _CHUNK, _CHUNK)]],
            rows[k % _NBUF], gsem[k % _NBUF])

    def start_pos(c):
        return pltpu.async_copy(
            pos_hbm.at[pl.ds(pos_base + c * _CHUNK, _CHUNK)],
            pos_v[c % 2], psem[c % 2])

    pdesc = {0: start_pos(0)}
    for d in idesc:
        d.wait()
    gdesc = {0: start_gather(0)}
    odesc = {}
    for k in range(_NG):
        c, b = divmod(k, _B)
        if b == 0 and c + 1 < _PC:
            pdesc[c + 1] = start_pos(c + 1)
        if b == 0:
            pdesc[c].wait()
        if k + 1 < _NG:
            if k - 2 >= 0:
                for d in odesc[k - 2]:
                    d.wait()
            gdesc[k + 1] = start_gather(k + 1)
        gdesc[k].wait()
        rv, pv = rows[k % _NBUF], pos_v[c % 2]
        out_base = b * _L_SEQ + pos_base + c * _CHUNK
        half = _CHUNK // 2

        def half_body(h, carry):
            def row_body(r, cr):
                for cc in range(_DL):
                    sl = pl.ds(cc * _LANES, _LANES)
                    plsc.addupdate(rv.at[r, sl], pv[r, sl])
                return cr

            lax.fori_loop(h * half, (h + 1) * half, row_body, 0)
            pltpu.async_copy(
                rv.at[pl.ds(h * half, half)],
                out_hbm.at[pl.ds(out_base + h * half, half)],
                osem[k % _NBUF])
            return carry

        lax.fori_loop(0, 2, half_body, 0)
        odesc[k] = [pltpu.make_async_copy(
            rv.at[pl.ds(h * half, half)],
            out_hbm.at[pl.ds(out_base + h * half, half)],
            osem[k % _NBUF]) for h in range(2)]
    for k in range(_NG - 3, _NG):
        for d in odesc[k]:
            d.wait()


@functools.partial(jax.jit, static_argnames=())
def kernel(x, emb_table, pos_encoding):
    seq_len = x.shape[1]
    idx = x.astype(jnp.int32)
    mesh = plsc.VectorSubcoreMesh(
        core_axis_name="c", subcore_axis_name="s",
        num_cores=_NC, num_subcores=_NS,
    )
    out = pl.kernel(
        _emb_body,
        out_type=jax.ShapeDtypeStruct((_B * _L_SEQ, _D), jnp.float32),
        mesh=mesh,
        scratch_types=(
            [pltpu.VMEM((_B, _POS_PER_W), jnp.int32)]
            + [pltpu.VMEM((_CHUNK, _D), jnp.float32)] * 2
            + [pltpu.VMEM((_CHUNK, _D), jnp.float32)] * _NBUF
            + [pltpu.SemaphoreType.DMA] * 8
        ),
    )(idx, emb_table, pos_encoding)
    return out.reshape(_B, seq_len, _D)
